# Initial kernel scaffold; baseline (speedup 1.0000x reference)
import jax
import jax.numpy as jnp
from jax.experimental import pallas as pl


def kernel(feats, centers, labels, camids, epoch):
    raise NotImplementedError("write your pallas kernel here")


# speedup vs baseline: 25.4393x; 25.4393x over previous
import jax
import jax.numpy as jnp
from jax.experimental import pallas as pl
from jax.experimental.pallas import tpu as pltpu

_B = 64
_D = 256
_L = 2000
_M = 16
_N = _L * _M
_T = 0.07
_HARD_K = 50
_LAMDA = 0.5
_BLK = 3200
_NB = _N // _BLK


def _loss_kernel(feats_ref, lab_ref, cam_ref, cen_ref, out_ref, acc_ref):
    i = pl.program_id(0)

    @pl.when(i == 0)
    def _init():
        acc_ref[...] = jnp.zeros_like(acc_ref)

    f = feats_ref[...]
    x = f / jnp.sqrt(jnp.sum(f * f, axis=1, keepdims=True))
    c = cen_ref[...]
    s = jax.lax.dot_general(x, c, (((1,), (1,)), ((), ())),
                            preferred_element_type=jnp.float32)
    e = jnp.exp(s / _T)

    lab = lab_ref[...]
    cam = cam_ref[...]
    j = jax.lax.broadcasted_iota(jnp.int32, (_B, _BLK), 1)
    g = j + i * _BLK
    lab16 = lab * _M

    f32 = jnp.float32
    cam_mask = (jnp.bitwise_and(g, _M - 1) == cam).astype(f32)
    pos_mask = ((g >= lab16) & (g < lab16 + _M)).astype(f32)
    up_mask = (g == lab16 + cam).astype(f32)
    hard_mask = (((g < lab16) & (g < _HARD_K)) |
                 ((g >= lab16 + _M) & (g < _HARD_K + _M))).astype(f32)

    upd = jnp.concatenate(
        [jnp.sum(e * cam_mask, axis=1, keepdims=True),
         jnp.sum(e * pos_mask, axis=1, keepdims=True),
         jnp.sum(e * up_mask, axis=1, keepdims=True),
         jnp.sum(e * hard_mask, axis=1, keepdims=True)], axis=1)
    acc_ref[...] += upd

    @pl.when(i == _NB - 1)
    def _finish():
        acc = acc_ref[...]
        log_up = jnp.log(acc[:, 2:3])
        log_fd = jnp.log(acc[:, 0:1])
        log_pd = jnp.log(acc[:, 1:2] + acc[:, 3:4])
        intra = -jnp.sum(log_up - log_fd)
        inter = -jnp.sum(log_up - log_pd)
        out_ref[...] = jnp.concatenate(
            [intra.reshape(1, 1), (_LAMDA * inter).reshape(1, 1)], axis=1)


def kernel(feats, centers, labels, camids, epoch):
    lab = labels.reshape(_B, 1).astype(jnp.int32)
    cam = camids.reshape(_B, 1).astype(jnp.int32)
    out = pl.pallas_call(
        _loss_kernel,
        grid=(_NB,),
        in_specs=[
            pl.BlockSpec((_B, _D), lambda i: (0, 0)),
            pl.BlockSpec((_B, 1), lambda i: (0, 0)),
            pl.BlockSpec((_B, 1), lambda i: (0, 0)),
            pl.BlockSpec((_BLK, _D), lambda i: (i, 0)),
        ],
        out_specs=pl.BlockSpec((1, 2), lambda i: (0, 0)),
        out_shape=jax.ShapeDtypeStruct((1, 2), jnp.float32),
        scratch_shapes=[pltpu.VMEM((_B, 4), jnp.float32)],
        compiler_params=pltpu.CompilerParams(
            dimension_semantics=("arbitrary",)),
    )(feats, lab, cam, centers)
    gate = (jnp.asarray(epoch) >= 5).astype(jnp.float32)
    return out.reshape(2) * gate


# speedup vs baseline: 27.4985x; 1.0809x over previous
import jax
import jax.numpy as jnp
from jax.experimental import pallas as pl
from jax.experimental.pallas import tpu as pltpu

_B = 64
_D = 256
_L = 2000
_M = 16
_N = _L * _M
_T = 0.07
_HARD_K = 50
_LAMDA = 0.5
_BLK = 3200
_NB = _N // _BLK


def _loss_kernel(feats_ref, lab_ref, cam_ref, cen_ref, out_ref,
                 x_ref, camm_ref, jdiv_ref, acc_ref, hard_ref):
    i = pl.program_id(0)
    lab = lab_ref[...]
    cam = cam_ref[...]

    @pl.when(i == 0)
    def _init():
        f = feats_ref[...]
        x_ref[...] = f / jnp.sqrt(jnp.sum(f * f, axis=1, keepdims=True))
        j = jax.lax.broadcasted_iota(jnp.int32, (_B, _BLK), 1)
        camm_ref[...] = (jnp.bitwise_and(j, _M - 1) == cam).astype(jnp.float32)
        jdiv_ref[...] = jax.lax.shift_right_logical(j, 4)
        acc_ref[...] = jnp.zeros_like(acc_ref)
        jh = jax.lax.broadcasted_iota(jnp.int32, (_B, 128), 1)
        lab16 = lab * _M
        hmask = (((jh < lab16) & (jh < _HARD_K)) |
                 ((jh >= lab16 + _M) & (jh < _HARD_K + _M)))
        hard_ref[...] = jnp.where(hmask, 1.0, 0.0)

    c = cen_ref[...]
    s = jax.lax.dot_general(x_ref[...], c, (((1,), (1,)), ((), ())),
                            preferred_element_type=jnp.float32)
    e = jnp.exp(s * (1.0 / _T))

    ecam = e * camm_ref[...]
    pos_mask = jdiv_ref[...] == (lab - i * (_BLK // _M))
    zero = jnp.zeros_like(e)
    fd = jnp.sum(ecam, axis=1, keepdims=True)
    pd = jnp.sum(jnp.where(pos_mask, e, zero), axis=1, keepdims=True)
    up = jnp.sum(jnp.where(pos_mask, ecam, zero), axis=1, keepdims=True)
    hscale = jnp.where(i == 0, 1.0, 0.0)
    hd = jnp.sum(e[:, :128] * hard_ref[...], axis=1, keepdims=True) * hscale
    acc_ref[...] += jnp.concatenate([fd, pd, up, hd], axis=1)

    @pl.when(i == _NB - 1)
    def _finish():
        acc = acc_ref[...]
        log_up = jnp.log(acc[:, 2:3])
        log_fd = jnp.log(acc[:, 0:1])
        log_pd = jnp.log(acc[:, 1:2] + acc[:, 3:4])
        intra = -jnp.sum(log_up - log_fd)
        inter = -jnp.sum(log_up - log_pd)
        out_ref[...] = jnp.concatenate(
            [intra.reshape(1, 1), (_LAMDA * inter).reshape(1, 1)], axis=1)


def kernel(feats, centers, labels, camids, epoch):
    lab = labels.reshape(_B, 1).astype(jnp.int32)
    cam = camids.reshape(_B, 1).astype(jnp.int32)
    out = pl.pallas_call(
        _loss_kernel,
        grid=(_NB,),
        in_specs=[
            pl.BlockSpec((_B, _D), lambda i: (0, 0)),
            pl.BlockSpec((_B, 1), lambda i: (0, 0)),
            pl.BlockSpec((_B, 1), lambda i: (0, 0)),
            pl.BlockSpec((_BLK, _D), lambda i: (i, 0)),
        ],
        out_specs=pl.BlockSpec((1, 2), lambda i: (0, 0)),
        out_shape=jax.ShapeDtypeStruct((1, 2), jnp.float32),
        scratch_shapes=[
            pltpu.VMEM((_B, _D), jnp.float32),
            pltpu.VMEM((_B, _BLK), jnp.float32),
            pltpu.VMEM((_B, _BLK), jnp.int32),
            pltpu.VMEM((_B, 4), jnp.float32),
            pltpu.VMEM((_B, 128), jnp.float32),
        ],
        compiler_params=pltpu.CompilerParams(
            dimension_semantics=("arbitrary",)),
    )(feats, lab, cam, centers)
    gate = (jnp.asarray(epoch) >= 5).astype(jnp.float32)
    return out.reshape(2) * gate


# speedup vs baseline: 30.2612x; 1.1005x over previous
import jax
import jax.numpy as jnp
from jax.experimental import pallas as pl
from jax.experimental.pallas import tpu as pltpu

_B = 64
_D = 256
_L = 2000
_M = 16
_N = _L * _M
_T = 0.07
_HARD_K = 50
_LAMDA = 0.5
_BLK = 6400
_NB = _N // _BLK


def _loss_kernel(feats_ref, lab_ref, cam_ref, cen_ref, out_ref,
                 x_ref, camm_ref, jdiv_ref, acc_ref, hard_ref):
    i = pl.program_id(0)
    lab = lab_ref[...]
    cam = cam_ref[...]

    @pl.when(i == 0)
    def _init():
        f = feats_ref[...]
        x_ref[...] = f / jnp.sqrt(jnp.sum(f * f, axis=1, keepdims=True))
        j = jax.lax.broadcasted_iota(jnp.int32, (_B, _BLK), 1)
        camm_ref[...] = (jnp.bitwise_and(j, _M - 1) == cam).astype(jnp.float32)
        jdiv_ref[...] = jax.lax.shift_right_logical(j, 4)
        acc_ref[...] = jnp.zeros_like(acc_ref)
        jh = jax.lax.broadcasted_iota(jnp.int32, (_B, 128), 1)
        lab16 = lab * _M
        hmask = (((jh < lab16) & (jh < _HARD_K)) |
                 ((jh >= lab16 + _M) & (jh < _HARD_K + _M)))
        hard_ref[...] = jnp.where(hmask, 1.0, 0.0)

    c = cen_ref[...]
    s = jax.lax.dot_general(x_ref[...], c, (((1,), (1,)), ((), ())),
                            preferred_element_type=jnp.float32)
    e = jnp.exp(s * (1.0 / _T))

    ecam = e * camm_ref[...]
    pos_mask = jdiv_ref[...] == (lab - i * (_BLK // _M))
    zero = jnp.zeros_like(e)
    fd = jnp.sum(ecam, axis=1, keepdims=True)
    pd = jnp.sum(jnp.where(pos_mask, e, zero), axis=1, keepdims=True)
    up = jnp.sum(jnp.where(pos_mask, ecam, zero), axis=1, keepdims=True)
    hscale = jnp.where(i == 0, 1.0, 0.0)
    hd = jnp.sum(e[:, :128] * hard_ref[...], axis=1, keepdims=True) * hscale
    acc_ref[...] += jnp.concatenate([fd, pd, up, hd], axis=1)

    @pl.when(i == _NB - 1)
    def _finish():
        acc = acc_ref[...]
        log_up = jnp.log(acc[:, 2:3])
        log_fd = jnp.log(acc[:, 0:1])
        log_pd = jnp.log(acc[:, 1:2] + acc[:, 3:4])
        intra = -jnp.sum(log_up - log_fd)
        inter = -jnp.sum(log_up - log_pd)
        out_ref[...] = jnp.concatenate(
            [intra.reshape(1, 1), (_LAMDA * inter).reshape(1, 1)], axis=1)


def kernel(feats, centers, labels, camids, epoch):
    lab = labels.reshape(_B, 1).astype(jnp.int32)
    cam = camids.reshape(_B, 1).astype(jnp.int32)
    out = pl.pallas_call(
        _loss_kernel,
        grid=(_NB,),
        in_specs=[
            pl.BlockSpec((_B, _D), lambda i: (0, 0)),
            pl.BlockSpec((_B, 1), lambda i: (0, 0)),
            pl.BlockSpec((_B, 1), lambda i: (0, 0)),
            pl.BlockSpec((_BLK, _D), lambda i: (i, 0)),
        ],
        out_specs=pl.BlockSpec((1, 2), lambda i: (0, 0)),
        out_shape=jax.ShapeDtypeStruct((1, 2), jnp.float32),
        scratch_shapes=[
            pltpu.VMEM((_B, _D), jnp.float32),
            pltpu.VMEM((_B, _BLK), jnp.float32),
            pltpu.VMEM((_B, _BLK), jnp.int32),
            pltpu.VMEM((_B, 4), jnp.float32),
            pltpu.VMEM((_B, 128), jnp.float32),
        ],
        compiler_params=pltpu.CompilerParams(
            dimension_semantics=("arbitrary",)),
    )(feats, lab, cam, centers)
    gate = (jnp.asarray(epoch) >= 5).astype(jnp.float32)
    return out.reshape(2) * gate


# speedup vs baseline: 30.6149x; 1.0117x over previous
import jax
import jax.numpy as jnp
from jax.experimental import pallas as pl
from jax.experimental.pallas import tpu as pltpu

_B = 64
_D = 256
_L = 2000
_M = 16
_N = _L * _M
_T = 0.07
_HARD_K = 50
_LAMDA = 0.5
_BLK = 16000
_NB = _N // _BLK


def _loss_kernel(feats_ref, lab_ref, cam_ref, cen_ref, out_ref,
                 x_ref, camm_ref, jdiv_ref, acc_ref, hard_ref):
    i = pl.program_id(0)
    lab = lab_ref[...]
    cam = cam_ref[...]

    @pl.when(i == 0)
    def _init():
        f = feats_ref[...]
        x_ref[...] = f / jnp.sqrt(jnp.sum(f * f, axis=1, keepdims=True))
        j = jax.lax.broadcasted_iota(jnp.int32, (_B, _BLK), 1)
        camm_ref[...] = (jnp.bitwise_and(j, _M - 1) == cam).astype(jnp.float32)
        jdiv_ref[...] = jax.lax.shift_right_logical(j, 4)
        acc_ref[...] = jnp.zeros_like(acc_ref)
        jh = jax.lax.broadcasted_iota(jnp.int32, (_B, 128), 1)
        lab16 = lab * _M
        hmask = (((jh < lab16) & (jh < _HARD_K)) |
                 ((jh >= lab16 + _M) & (jh < _HARD_K + _M)))
        hard_ref[...] = jnp.where(hmask, 1.0, 0.0)

    c = cen_ref[...]
    s = jax.lax.dot_general(x_ref[...], c, (((1,), (1,)), ((), ())),
                            preferred_element_type=jnp.float32)
    e = jnp.exp(s * (1.0 / _T))

    ecam = e * camm_ref[...]
    pos_mask = jdiv_ref[...] == (lab - i * (_BLK // _M))
    zero = jnp.zeros_like(e)
    fd = jnp.sum(ecam, axis=1, keepdims=True)
    pd = jnp.sum(jnp.where(pos_mask, e, zero), axis=1, keepdims=True)
    up = jnp.sum(jnp.where(pos_mask, ecam, zero), axis=1, keepdims=True)
    hscale = jnp.where(i == 0, 1.0, 0.0)
    hd = jnp.sum(e[:, :128] * hard_ref[...], axis=1, keepdims=True) * hscale
    acc_ref[...] += jnp.concatenate([fd, pd, up, hd], axis=1)

    @pl.when(i == _NB - 1)
    def _finish():
        acc = acc_ref[...]
        log_up = jnp.log(acc[:, 2:3])
        log_fd = jnp.log(acc[:, 0:1])
        log_pd = jnp.log(acc[:, 1:2] + acc[:, 3:4])
        intra = -jnp.sum(log_up - log_fd)
        inter = -jnp.sum(log_up - log_pd)
        out_ref[...] = jnp.concatenate(
            [intra.reshape(1, 1), (_LAMDA * inter).reshape(1, 1)], axis=1)


def kernel(feats, centers, labels, camids, epoch):
    lab = labels.reshape(_B, 1).astype(jnp.int32)
    cam = camids.reshape(_B, 1).astype(jnp.int32)
    out = pl.pallas_call(
        _loss_kernel,
        grid=(_NB,),
        in_specs=[
            pl.BlockSpec((_B, _D), lambda i: (0, 0)),
            pl.BlockSpec((_B, 1), lambda i: (0, 0)),
            pl.BlockSpec((_B, 1), lambda i: (0, 0)),
            pl.BlockSpec((_BLK, _D), lambda i: (i, 0)),
        ],
        out_specs=pl.BlockSpec((1, 2), lambda i: (0, 0)),
        out_shape=jax.ShapeDtypeStruct((1, 2), jnp.float32),
        scratch_shapes=[
            pltpu.VMEM((_B, _D), jnp.float32),
            pltpu.VMEM((_B, _BLK), jnp.float32),
            pltpu.VMEM((_B, _BLK), jnp.int32),
            pltpu.VMEM((_B, 4), jnp.float32),
            pltpu.VMEM((_B, 128), jnp.float32),
        ],
        compiler_params=pltpu.CompilerParams(
            dimension_semantics=("arbitrary",)),
    )(feats, lab, cam, centers)
    gate = (jnp.asarray(epoch) >= 5).astype(jnp.float32)
    return out.reshape(2) * gate
